# baseline (speedup 1.0000x reference)
import functools

import jax
import jax.numpy as jnp
from jax import lax
from jax.experimental import pallas as pl
from jax.experimental.pallas import tpu as pltpu
from jax.experimental.pallas import tpu_sc as plsc

V1, V2, V3 = 50, 40, 50
E1, E2, E3 = 4, 4, 4
R1, R2 = 16, 16
VOCAB = V1 * V2 * V3
EMB = E1 * E2 * E3
TROW = 2 * EMB
NV12 = V1 * V2
KDIM = E1 * E2 * R2
NCOLS = V3 * EMB
ROWBLK = 400
OROWS = ROWBLK * NCOLS // TROW

NC, NS = 2, 16
NW = NC * NS
NROW, NCOL = 4096, 26
B_TOTAL = NROW * NCOL
B_PER_W = B_TOTAL // NW
KCH = 128
NCH = B_PER_W // KCH
L = 16


def _w12_body(c0_ref, c1_ref, out_ref):
    m0 = c0_ref[...].reshape(V1 * E1, R1)
    c1 = c1_ref[...].reshape(R1, V2 * E2 * R2)
    p = jnp.dot(m0, c1, preferred_element_type=jnp.float32)
    w12 = p.reshape(V1, E1, V2, E2 * R2).transpose(0, 2, 1, 3)
    out_ref[...] = w12.reshape(NV12, KDIM)


def _table_body(w12_ref, c2f_ref, out_ref, bd_ref):
    @pl.when(pl.program_id(0) == 0)
    def _():
        c2f = c2f_ref[...]
        pr = lax.broadcasted_iota(jnp.int32, (KDIM, R2), 0)
        rc = lax.broadcasted_iota(jnp.int32, (KDIM, R2), 1)
        oh_r = jnp.where(rc == pr % R2, 1.0, 0.0).astype(jnp.float32)
        cc = lax.broadcasted_iota(jnp.int32, (V3 * E3, NCOLS), 0)
        qq = lax.broadcasted_iota(jnp.int32, (V3 * E3, NCOLS), 1)
        hit = cc == (qq // EMB) * E3 + qq % E3
        oh_c = jnp.where(hit, 1.0, 0.0).astype(jnp.float32)
        v = jnp.dot(jnp.dot(oh_r, c2f, preferred_element_type=jnp.float32),
                    oh_c, preferred_element_type=jnp.float32)
        p2 = lax.broadcasted_iota(jnp.int32, (KDIM, NCOLS), 0)
        q2 = lax.broadcasted_iota(jnp.int32, (KDIM, NCOLS), 1)
        bd_ref[...] = jnp.where((p2 // R2) == ((q2 % EMB) // E3), v, 0.0)

    res = jnp.dot(w12_ref[...], bd_ref[...], preferred_element_type=jnp.float32)
    out_ref[...] = res.reshape(OROWS, TROW)


def _build_table(core0, core1, core2):
    w12 = pl.pallas_call(
        _w12_body,
        out_shape=jax.ShapeDtypeStruct((NV12, KDIM), jnp.float32),
    )(core0, core1)
    c2f = core2.reshape(R2, V3 * E3)
    table = pl.pallas_call(
        _table_body,
        grid=(NV12 // ROWBLK,),
        in_specs=[
            pl.BlockSpec((ROWBLK, KDIM), lambda i: (i, 0)),
            pl.BlockSpec((R2, V3 * E3), lambda i: (0, 0)),
        ],
        out_specs=pl.BlockSpec((OROWS, TROW), lambda i: (i, 0)),
        out_shape=jax.ShapeDtypeStruct((VOCAB // 2, TROW), jnp.float32),
        scratch_shapes=[pltpu.VMEM((KDIM, NCOLS), jnp.float32)],
    )(w12, c2f)
    return table


def _gather_rows(table, idx3):
    mesh = plsc.VectorSubcoreMesh(core_axis_name="c", subcore_axis_name="s")

    @functools.partial(
        pl.kernel,
        mesh=mesh,
        out_type=jax.ShapeDtypeStruct((NCOL * NROW, TROW), jnp.float32),
        scratch_types=[
            pltpu.VMEM((NCH, KCH), jnp.int32),
            pltpu.VMEM((KCH, TROW), jnp.float32),
            pltpu.VMEM((KCH, TROW), jnp.float32),
            pltpu.SemaphoreType.DMA,
            pltpu.SemaphoreType.DMA,
        ],
    )
    def gather_k(table_hbm, idx_hbm, out_hbm, idx_v, rows_a, rows_b, sem_a, sem_b):
        wid = lax.axis_index("s") * NC + lax.axis_index("c")
        pltpu.sync_copy(idx_hbm.at[wid], idx_v)
        bufs = (rows_a, rows_b)
        sems = (sem_a, sem_b)

        pltpu.make_async_copy(table_hbm.at[idx_v.at[0]], rows_a, sem_a).start()

        def body(ch0, carry):
            for b in range(2):
                ch = ch0 + b

                @pl.when(ch + 1 < NCH)
                def _():
                    pltpu.make_async_copy(
                        table_hbm.at[idx_v.at[ch + 1]], bufs[1 - b], sems[1 - b]
                    ).start()

                pltpu.make_async_copy(
                    table_hbm.at[idx_v.at[ch]], bufs[b], sems[b]
                ).wait()
                pltpu.sync_copy(bufs[b],
                                out_hbm.at[pl.ds(ch * NROW + wid * KCH, KCH)])
            return carry

        lax.fori_loop(0, NCH // 2, lambda i, c: body(i * 2, c), 0, unroll=False)

    return gather_k(table, idx3)


def _format_body(in_ref, par_ref, out_ref):
    i = pl.program_id(0)
    x = in_ref[...]
    pr = par_ref[pl.ds(i, 1), :]
    ce = lax.broadcasted_iota(jnp.int32, (TROW, TROW), 0)
    cq = lax.broadcasted_iota(jnp.int32, (TROW, TROW), 1)
    eye = jnp.where(ce == cq, 1.0, 0.0).astype(jnp.float32)
    xt = lax.dot_general(eye, x, (((1,), (1,)), ((), ())),
                         precision=lax.Precision.HIGHEST,
                         preferred_element_type=jnp.float32)
    out_ref[...] = jnp.where(pr > 0.5, xt[EMB:, :], xt[:EMB, :])


def _format_rows(rows, par2):
    return pl.pallas_call(
        _format_body,
        grid=(NCOL,),
        in_specs=[
            pl.BlockSpec((NROW, TROW), lambda i: (i, 0)),
            pl.BlockSpec((NCOL, NROW), lambda i: (0, 0)),
        ],
        out_specs=pl.BlockSpec((EMB, NROW), lambda i: (i, 0)),
        out_shape=jax.ShapeDtypeStruct((NCOL * EMB, NROW), jnp.float32),
    )(rows, par2)


def kernel(core0, core1, core2, x):
    table = _build_table(core0, core1, core2)
    xi = x.astype(jnp.int32)
    idx3 = (xi >> 1).reshape(NW, KCH, NCOL).transpose(0, 2, 1)
    par = (xi & 1).T.astype(jnp.float32)
    rows = _gather_rows(table, idx3)
    outt = _format_rows(rows, par)
    return outt.reshape(NCOL, EMB, NROW).transpose(2, 0, 1)


# speedup vs baseline: 1.1613x; 1.1613x over previous
import functools

import jax
import jax.numpy as jnp
from jax import lax
from jax.experimental import pallas as pl
from jax.experimental.pallas import tpu as pltpu
from jax.experimental.pallas import tpu_sc as plsc

V1, V2, V3 = 50, 40, 50
E1, E2, E3 = 4, 4, 4
R1, R2 = 16, 16
VOCAB = V1 * V2 * V3
EMB = E1 * E2 * E3
TROW = 2 * EMB
NV12 = V1 * V2
KDIM = E1 * E2 * R2
NCOLS = V3 * EMB
ROWBLK = 400
OROWS = ROWBLK * NCOLS // TROW

NC, NS = 2, 16
NW = NC * NS
NROW, NCOL = 4096, 26
B_TOTAL = NROW * NCOL
B_PER_W = B_TOTAL // NW
KCH = 128
NCH = B_PER_W // KCH
L = 16


def _w12_body(c0_ref, c1_ref, out_ref):
    m0 = c0_ref[...].reshape(V1 * E1, R1)
    c1 = c1_ref[...].reshape(R1, V2 * E2 * R2)
    p = jnp.dot(m0, c1, preferred_element_type=jnp.float32)
    w12 = p.reshape(V1, E1, V2, E2 * R2).transpose(0, 2, 1, 3)
    out_ref[...] = w12.reshape(NV12, KDIM)


def _table_body(w12_ref, c2f_ref, out_ref, bd_ref):
    @pl.when(pl.program_id(0) == 0)
    def _():
        c2f = c2f_ref[...]
        pr = lax.broadcasted_iota(jnp.int32, (KDIM, R2), 0)
        rc = lax.broadcasted_iota(jnp.int32, (KDIM, R2), 1)
        oh_r = jnp.where(rc == pr % R2, 1.0, 0.0).astype(jnp.float32)
        cc = lax.broadcasted_iota(jnp.int32, (V3 * E3, NCOLS), 0)
        qq = lax.broadcasted_iota(jnp.int32, (V3 * E3, NCOLS), 1)
        hit = cc == (qq // EMB) * E3 + qq % E3
        oh_c = jnp.where(hit, 1.0, 0.0).astype(jnp.float32)
        v = jnp.dot(jnp.dot(oh_r, c2f, preferred_element_type=jnp.float32),
                    oh_c, preferred_element_type=jnp.float32)
        p2 = lax.broadcasted_iota(jnp.int32, (KDIM, NCOLS), 0)
        q2 = lax.broadcasted_iota(jnp.int32, (KDIM, NCOLS), 1)
        bd_ref[...] = jnp.where((p2 // R2) == ((q2 % EMB) // E3), v, 0.0)

    res = jnp.dot(w12_ref[...], bd_ref[...], preferred_element_type=jnp.float32)
    out_ref[...] = res.reshape(OROWS, TROW)


def _build_table(core0, core1, core2):
    w12 = pl.pallas_call(
        _w12_body,
        out_shape=jax.ShapeDtypeStruct((NV12, KDIM), jnp.float32),
    )(core0, core1)
    c2f = core2.reshape(R2, V3 * E3)
    table = pl.pallas_call(
        _table_body,
        grid=(NV12 // ROWBLK,),
        in_specs=[
            pl.BlockSpec((ROWBLK, KDIM), lambda i: (i, 0)),
            pl.BlockSpec((R2, V3 * E3), lambda i: (0, 0)),
        ],
        out_specs=pl.BlockSpec((OROWS, TROW), lambda i: (i, 0)),
        out_shape=jax.ShapeDtypeStruct((VOCAB // 2, TROW), jnp.float32),
        scratch_shapes=[pltpu.VMEM((KDIM, NCOLS), jnp.float32)],
    )(w12, c2f)
    return table


def _gather_rows(table, idx3):
    mesh = plsc.VectorSubcoreMesh(core_axis_name="c", subcore_axis_name="s")

    @functools.partial(
        pl.kernel,
        mesh=mesh,
        out_type=jax.ShapeDtypeStruct((NCOL * NROW, TROW), jnp.float32),
        scratch_types=[
            pltpu.VMEM((NCH, KCH), jnp.int32),
            pltpu.VMEM((KCH, TROW), jnp.float32),
            pltpu.VMEM((KCH, TROW), jnp.float32),
            pltpu.SemaphoreType.DMA,
            pltpu.SemaphoreType.DMA,
        ],
    )
    def gather_k(table_hbm, idx_hbm, out_hbm, idx_v, rows_a, rows_b, sem_a, sem_b):
        wid = lax.axis_index("s") * NC + lax.axis_index("c")
        pltpu.sync_copy(idx_hbm.at[wid], idx_v)
        bufs = (rows_a, rows_b)
        sems = (sem_a, sem_b)

        pltpu.make_async_copy(table_hbm.at[idx_v.at[0]], rows_a, sem_a).start()

        def body(ch0, carry):
            for b in range(2):
                ch = ch0 + b

                @pl.when(ch + 1 < NCH)
                def _():
                    pltpu.make_async_copy(
                        table_hbm.at[idx_v.at[ch + 1]], bufs[1 - b], sems[1 - b]
                    ).start()

                pltpu.make_async_copy(
                    table_hbm.at[idx_v.at[ch]], bufs[b], sems[b]
                ).wait()
                pltpu.sync_copy(bufs[b],
                                out_hbm.at[pl.ds(ch * NROW + wid * KCH, KCH)])
            return carry

        lax.fori_loop(0, NCH // 2, lambda i, c: body(i * 2, c), 0, unroll=False)

    return gather_k(table, idx3)


def _format_body(in_ref, par_ref, out_ref):
    i = pl.program_id(0)
    x = in_ref[...]
    pr = par_ref[pl.ds(i, 1), :]
    ce = lax.broadcasted_iota(jnp.int32, (TROW, TROW), 0)
    cq = lax.broadcasted_iota(jnp.int32, (TROW, TROW), 1)
    eye = jnp.where(ce == cq, 1.0, 0.0).astype(jnp.float32)
    xt = lax.dot_general(eye, x, (((1,), (1,)), ((), ())),
                         preferred_element_type=jnp.float32)
    out_ref[...] = jnp.where(pr > 0.5, xt[EMB:, :], xt[:EMB, :])


def _format_rows(rows, par2):
    return pl.pallas_call(
        _format_body,
        grid=(NCOL,),
        in_specs=[
            pl.BlockSpec((NROW, TROW), lambda i: (i, 0)),
            pl.BlockSpec((NCOL, NROW), lambda i: (0, 0)),
        ],
        out_specs=pl.BlockSpec((EMB, NROW), lambda i: (i, 0)),
        out_shape=jax.ShapeDtypeStruct((NCOL * EMB, NROW), jnp.float32),
    )(rows, par2)


def kernel(core0, core1, core2, x):
    table = _build_table(core0, core1, core2)
    xi = x.astype(jnp.int32)
    idx3 = (xi >> 1).reshape(NW, KCH, NCOL).transpose(0, 2, 1)
    par = (xi & 1).T.astype(jnp.float32)
    rows = _gather_rows(table, idx3)
    outt = _format_rows(rows, par)
    return outt.reshape(NCOL, EMB, NROW).transpose(2, 0, 1)
